# baseline (speedup 1.0000x reference)
import functools

import jax
import jax.numpy as jnp
from jax import lax
from jax.experimental import pallas as pl
from jax.experimental.pallas import tpu as pltpu
from jax.experimental.pallas import tpu_sc as plsc

_PAD_D = 16
_NUM_WORKERS = 32
_IDX_CHUNK = 128
_ROW_BLOCK = 256


def _full16(v):
    return jnp.full((16,), v, jnp.int32)


def _sc_gather_components(table, idx2d, poly16):
    e_total = poly16.shape[0]
    e_per_w = e_total // _NUM_WORKERS
    rows_per_w = 4 * e_per_w
    streams = rows_per_w // _IDX_CHUNK
    lane_grp = e_per_w // 128
    mesh = plsc.VectorSubcoreMesh(core_axis_name="c", subcore_axis_name="s")

    @functools.partial(
        pl.kernel,
        mesh=mesh,
        out_type=jax.ShapeDtypeStruct((24, e_total // 128, 128), jnp.float32),
        compiler_params=pltpu.CompilerParams(
            use_tc_tiling_on_sc=False, needs_layout_passes=False),
        scratch_types=[
            pltpu.VMEM((streams, _IDX_CHUNK), jnp.int32),
            pltpu.VMEM((rows_per_w, _PAD_D), jnp.float32),
            pltpu.VMEM((e_per_w, _PAD_D), jnp.float32),
            pltpu.VMEM((24, lane_grp, 128), jnp.float32),
            pltpu.SemaphoreType.DMA,
        ],
    )
    def gather_kernel(table_hbm, idx_hbm, poly_hbm, out_hbm,
                      idx_v, rows_v, poly_v, comp_v, sem):
        wid = lax.axis_index("s") * 2 + lax.axis_index("c")
        pltpu.sync_copy(idx_hbm.at[pl.ds(wid * streams, streams)], idx_v)
        copies = [pltpu.async_copy(
            poly_hbm.at[pl.ds(wid * e_per_w, e_per_w)], poly_v, sem)]
        for j in range(streams):
            copies.append(
                pltpu.async_copy(
                    table_hbm.at[idx_v.at[j]],
                    rows_v.at[pl.ds(j * _IDX_CHUNK, _IDX_CHUNK)],
                    sem,
                )
            )
        for c in copies:
            c.wait()

        iot = lax.iota(jnp.int32, 16)
        iot4 = iot * 4
        for j4 in range(lane_grp):
            @pl.loop(0, 128, step=16)
            def _(m, j4=j4):
                g = j4 * 128 + m
                for f in range(4):
                    ridx = iot4 + (g * 4 + f)
                    for t in range(3):
                        comp_v[f * 3 + t, j4, pl.ds(m, 16)] = plsc.load_gather(
                            rows_v, [ridx, _full16(t)])
                    pidx = iot + g
                    for l in range(3):
                        comp_v[12 + f * 3 + l, j4, pl.ds(m, 16)] = (
                            plsc.load_gather(
                                poly_v, [pidx, _full16(4 * f + l)]))

        pltpu.sync_copy(
            comp_v, out_hbm.at[:, pl.ds(wid * lane_grp, lane_grp), :])

    return gather_kernel(table, idx2d, poly16)


def _elastic_body(c_ref, meas_ref, lam_ref, mu_ref, out_ref):
    a = [c_ref[i] for i in range(24)]
    f_mat = [[None] * 3 for _ in range(3)]
    for t in range(3):
        for l in range(3):
            acc = a[0 * 3 + t] * a[12 + 0 * 3 + l]
            for f in range(1, 4):
                acc += a[f * 3 + t] * a[12 + f * 3 + l]
            f_mat[t][l] = acc
    ic = f_mat[0][0] * f_mat[0][0]
    for t in range(3):
        for l in range(3):
            if t or l:
                ic += f_mat[t][l] * f_mat[t][l]
    det = (
        f_mat[0][0] * (f_mat[1][1] * f_mat[2][2] - f_mat[1][2] * f_mat[2][1])
        - f_mat[0][1] * (f_mat[1][0] * f_mat[2][2] - f_mat[1][2] * f_mat[2][0])
        + f_mat[0][2] * (f_mat[1][0] * f_mat[2][1] - f_mat[1][1] * f_mat[2][0])
    )
    meas, lam_v, mu_v = meas_ref[...], lam_ref[...], mu_ref[...]
    alpha = 0.75 * mu_v / lam_v + 1.0
    ic_ver = jnp.maximum(ic + 1.0, 0.0) + 1e-30
    dens = (
        0.5 * mu_v * (ic - 3.0)
        + 0.5 * lam_v * (det - alpha) ** 2
        - 0.5 * mu_v * jnp.log(ic_ver)
    )
    out_ref[0, 0] = jnp.sum(dens * meas)


def _kinetic_body(m_ref, dt_ref, dn_ref, out_ref):
    i = pl.program_id(0)

    @pl.when(i == 0)
    def _():
        out_ref[0, 0] = 0.0

    m = m_ref[...]
    acc = jnp.float32(0.0)
    for k in range(3):
        s = jnp.sum(m * dt_ref[k : k + 1, :], axis=1, keepdims=True)
        acc += jnp.sum(s * dn_ref[:, k : k + 1])
    out_ref[0, 0] += acc


def kernel(position, time_step, state_position, velocity,
           external_acceleration, int_density_matrix, elements, polynomials,
           measure, lam, mu):
    n = position.shape[0]
    e = elements.shape[0]
    f32 = jnp.float32
    dt = jnp.asarray(time_step, f32)
    coeff = 0.5 / (dt * dt)

    table = jnp.zeros((n, _PAD_D), f32).at[:, :3].set(position)
    idx2d = elements.reshape(-1, _IDX_CHUNK).astype(jnp.int32)
    poly16 = polynomials.reshape(e, _PAD_D)
    comp = _sc_gather_components(table, idx2d, poly16)

    eb = e // 128
    elastic = pl.pallas_call(
        _elastic_body,
        out_shape=jax.ShapeDtypeStruct((1, 1), f32),
        in_specs=[
            pl.BlockSpec((24, eb, 128), lambda: (0, 0, 0)),
            pl.BlockSpec((eb, 128), lambda: (0, 0)),
            pl.BlockSpec((eb, 128), lambda: (0, 0)),
            pl.BlockSpec((eb, 128), lambda: (0, 0)),
        ],
        out_specs=pl.BlockSpec(memory_space=pltpu.SMEM),
    )(comp, measure.reshape(eb, 128),
      lam.reshape(eb, 128), mu.reshape(eb, 128))[0, 0]

    y = state_position + velocity * dt + external_acceleration * (dt * dt)
    delta = (position - y).astype(f32)
    dt_t = jnp.zeros((8, n), f32).at[:3, :].set(delta.T)
    dn = jnp.zeros((n, 8), f32).at[:, :3].set(delta)

    kin_raw = pl.pallas_call(
        _kinetic_body,
        grid=(n // _ROW_BLOCK,),
        out_shape=jax.ShapeDtypeStruct((1, 1), f32),
        in_specs=[
            pl.BlockSpec((_ROW_BLOCK, n), lambda i: (i, 0)),
            pl.BlockSpec((8, n), lambda i: (0, 0)),
            pl.BlockSpec((_ROW_BLOCK, 8), lambda i: (i, 0)),
        ],
        out_specs=pl.BlockSpec(memory_space=pltpu.SMEM),
    )(int_density_matrix, dt_t, dn)[0, 0]

    return (coeff * kin_raw + elastic).astype(f32)


# speedup vs baseline: 1.4889x; 1.4889x over previous
import functools

import jax
import jax.numpy as jnp
from jax import lax
from jax.experimental import pallas as pl
from jax.experimental.pallas import tpu as pltpu
from jax.experimental.pallas import tpu_sc as plsc

_PAD_D = 16
_NUM_WORKERS = 32
_IDX_CHUNK = 128
_ROW_BLOCK = 512


def _full16(v):
    return jnp.full((16,), v, jnp.int32)


def _sc_gather_components(table, idx_t):
    e_total = idx_t.shape[1]
    e_per_w = e_total // _NUM_WORKERS
    rows_per_w = 4 * e_per_w
    chunks = e_per_w // _IDX_CHUNK
    lane_grp = e_per_w // 128
    mesh = plsc.VectorSubcoreMesh(core_axis_name="c", subcore_axis_name="s")

    @functools.partial(
        pl.kernel,
        mesh=mesh,
        out_type=jax.ShapeDtypeStruct((12, e_total // 128, 128), jnp.float32),
        compiler_params=pltpu.CompilerParams(
            use_tc_tiling_on_sc=False, needs_layout_passes=False),
        scratch_types=[
            pltpu.VMEM((4, e_per_w), jnp.int32),
            pltpu.VMEM((rows_per_w, _PAD_D), jnp.float32),
            pltpu.VMEM((12, lane_grp, 128), jnp.float32),
            pltpu.SemaphoreType.DMA,
        ],
    )
    def gather_kernel(table_hbm, idx_hbm, out_hbm, idx_v, rows_v, comp_v, sem):
        wid = lax.axis_index("s") * 2 + lax.axis_index("c")
        base_e = wid * e_per_w
        for f in range(4):
            pltpu.sync_copy(idx_hbm.at[f, pl.ds(base_e, e_per_w)],
                            idx_v.at[f])
        copies = []
        for f in range(4):
            for c in range(4):
                copies.append(
                    pltpu.async_copy(
                        table_hbm.at[idx_v.at[f, pl.ds(c * _IDX_CHUNK,
                                                       _IDX_CHUNK)]],
                        rows_v.at[pl.ds((f * chunks + c) * _IDX_CHUNK,
                                        _IDX_CHUNK)],
                        sem,
                    )
                )
        for cp in copies:
            cp.wait()

        iot = lax.iota(jnp.int32, 16)
        for j4 in range(lane_grp):
            @pl.loop(0, 128, step=16)
            def _(m, j4=j4):
                g = j4 * 128 + m
                for f in range(4):
                    ridx = iot + (f * e_per_w + g)
                    for t in range(3):
                        comp_v[f * 3 + t, j4, pl.ds(m, 16)] = plsc.load_gather(
                            rows_v, [ridx, _full16(t)])

        pltpu.sync_copy(
            comp_v, out_hbm.at[:, pl.ds(wid * lane_grp, lane_grp), :])

    return gather_kernel(table, idx_t)


def _elastic_body(c_ref, p_ref, meas_ref, lam_ref, mu_ref, out_ref):
    a = [c_ref[i] for i in range(12)]
    b = [p_ref[i] for i in range(16)]
    f_mat = [[None] * 3 for _ in range(3)]
    for t in range(3):
        for l in range(3):
            acc = a[0 * 3 + t] * b[0 * 4 + l]
            for f in range(1, 4):
                acc += a[f * 3 + t] * b[f * 4 + l]
            f_mat[t][l] = acc
    ic = f_mat[0][0] * f_mat[0][0]
    for t in range(3):
        for l in range(3):
            if t or l:
                ic += f_mat[t][l] * f_mat[t][l]
    det = (
        f_mat[0][0] * (f_mat[1][1] * f_mat[2][2] - f_mat[1][2] * f_mat[2][1])
        - f_mat[0][1] * (f_mat[1][0] * f_mat[2][2] - f_mat[1][2] * f_mat[2][0])
        + f_mat[0][2] * (f_mat[1][0] * f_mat[2][1] - f_mat[1][1] * f_mat[2][0])
    )
    meas, lam_v, mu_v = meas_ref[...], lam_ref[...], mu_ref[...]
    alpha = 0.75 * mu_v / lam_v + 1.0
    ic_ver = jnp.maximum(ic + 1.0, 0.0) + 1e-30
    dens = (
        0.5 * mu_v * (ic - 3.0)
        + 0.5 * lam_v * (det - alpha) ** 2
        - 0.5 * mu_v * jnp.log(ic_ver)
    )
    out_ref[0, 0] = jnp.sum(dens * meas)


def _kinetic_body(m_ref, dt_ref, dn_ref, out_ref):
    i = pl.program_id(0)

    @pl.when(i == 0)
    def _():
        out_ref[0, 0] = 0.0

    m = m_ref[...]
    acc = jnp.float32(0.0)
    for k in range(3):
        s = jnp.sum(m * dt_ref[k : k + 1, :], axis=1, keepdims=True)
        acc += jnp.sum(s * dn_ref[:, k : k + 1])
    out_ref[0, 0] += acc


def kernel(position, time_step, state_position, velocity,
           external_acceleration, int_density_matrix, elements, polynomials,
           measure, lam, mu):
    n = position.shape[0]
    e = elements.shape[0]
    eb = e // 128
    f32 = jnp.float32
    dt = jnp.asarray(time_step, f32)
    coeff = 0.5 / (dt * dt)

    table = jnp.zeros((n, _PAD_D), f32).at[:, :3].set(position)
    idx_t = elements.T.astype(jnp.int32)
    comp = _sc_gather_components(table, idx_t)

    polyc = jnp.transpose(polynomials, (1, 2, 0)).reshape(16, eb, 128)

    elastic = pl.pallas_call(
        _elastic_body,
        out_shape=jax.ShapeDtypeStruct((1, 1), f32),
        in_specs=[
            pl.BlockSpec((12, eb, 128), lambda: (0, 0, 0)),
            pl.BlockSpec((16, eb, 128), lambda: (0, 0, 0)),
            pl.BlockSpec((eb, 128), lambda: (0, 0)),
            pl.BlockSpec((eb, 128), lambda: (0, 0)),
            pl.BlockSpec((eb, 128), lambda: (0, 0)),
        ],
        out_specs=pl.BlockSpec(memory_space=pltpu.SMEM),
    )(comp, polyc, measure.reshape(eb, 128),
      lam.reshape(eb, 128), mu.reshape(eb, 128))[0, 0]

    delta_t = (position.T - state_position.T - velocity.T * dt
               - external_acceleration.T * (dt * dt)).astype(f32)
    dt_t = jnp.zeros((8, n), f32).at[:3, :].set(delta_t)
    dn = jnp.zeros((n, 8), f32).at[:, :3].set(delta_t.T)

    kin_raw = pl.pallas_call(
        _kinetic_body,
        grid=(n // _ROW_BLOCK,),
        out_shape=jax.ShapeDtypeStruct((1, 1), f32),
        in_specs=[
            pl.BlockSpec((_ROW_BLOCK, n), lambda i: (i, 0)),
            pl.BlockSpec((8, n), lambda i: (0, 0)),
            pl.BlockSpec((_ROW_BLOCK, 8), lambda i: (i, 0)),
        ],
        out_specs=pl.BlockSpec(memory_space=pltpu.SMEM),
    )(int_density_matrix, dt_t, dn)[0, 0]

    return (coeff * kin_raw + elastic).astype(f32)


# speedup vs baseline: 1.5170x; 1.0189x over previous
import functools

import jax
import jax.numpy as jnp
from jax import lax
from jax.experimental import pallas as pl
from jax.experimental.pallas import tpu as pltpu
from jax.experimental.pallas import tpu_sc as plsc

_PAD_D = 8
_NUM_WORKERS = 32
_IDX_CHUNK = 128
_ROW_BLOCK = 512


def _full16(v):
    return jnp.full((16,), v, jnp.int32)


def _sc_gather_components(table, idx_t):
    e_total = idx_t.shape[1]
    e_per_w = e_total // _NUM_WORKERS
    rows_per_w = 4 * e_per_w
    chunks = e_per_w // _IDX_CHUNK
    lane_grp = e_per_w // 128
    mesh = plsc.VectorSubcoreMesh(core_axis_name="c", subcore_axis_name="s")

    @functools.partial(
        pl.kernel,
        mesh=mesh,
        out_type=jax.ShapeDtypeStruct((12, e_total // 128, 128), jnp.float32),
        compiler_params=pltpu.CompilerParams(
            use_tc_tiling_on_sc=False, needs_layout_passes=False),
        scratch_types=[
            pltpu.VMEM((4, e_per_w), jnp.int32),
            pltpu.VMEM((rows_per_w, _PAD_D), jnp.float32),
            pltpu.VMEM((12, lane_grp, 128), jnp.float32),
            pltpu.SemaphoreType.DMA,
        ],
    )
    def gather_kernel(table_hbm, idx_hbm, out_hbm, idx_v, rows_v, comp_v, sem):
        wid = lax.axis_index("s") * 2 + lax.axis_index("c")
        base_e = wid * e_per_w
        for f in range(4):
            pltpu.sync_copy(idx_hbm.at[f, pl.ds(base_e, e_per_w)],
                            idx_v.at[f])
        copies = []
        for f in range(4):
            for c in range(4):
                copies.append(
                    pltpu.async_copy(
                        table_hbm.at[idx_v.at[f, pl.ds(c * _IDX_CHUNK,
                                                       _IDX_CHUNK)]],
                        rows_v.at[pl.ds((f * chunks + c) * _IDX_CHUNK,
                                        _IDX_CHUNK)],
                        sem,
                    )
                )
        for cp in copies:
            cp.wait()

        iot = lax.iota(jnp.int32, 16)
        for j4 in range(lane_grp):
            @pl.loop(0, 128, step=16)
            def _(m, j4=j4):
                g = j4 * 128 + m
                for f in range(4):
                    ridx = iot + (f * e_per_w + g)
                    for t in range(3):
                        comp_v[f * 3 + t, j4, pl.ds(m, 16)] = plsc.load_gather(
                            rows_v, [ridx, _full16(t)])

        pltpu.sync_copy(
            comp_v, out_hbm.at[:, pl.ds(wid * lane_grp, lane_grp), :])

    return gather_kernel(table, idx_t)


def _elastic_body(c_ref, p_ref, meas_ref, lam_ref, mu_ref, out_ref):
    a = [c_ref[i] for i in range(12)]
    b = [p_ref[i] for i in range(16)]
    f_mat = [[None] * 3 for _ in range(3)]
    for t in range(3):
        for l in range(3):
            acc = a[0 * 3 + t] * b[0 * 4 + l]
            for f in range(1, 4):
                acc += a[f * 3 + t] * b[f * 4 + l]
            f_mat[t][l] = acc
    ic = f_mat[0][0] * f_mat[0][0]
    for t in range(3):
        for l in range(3):
            if t or l:
                ic += f_mat[t][l] * f_mat[t][l]
    det = (
        f_mat[0][0] * (f_mat[1][1] * f_mat[2][2] - f_mat[1][2] * f_mat[2][1])
        - f_mat[0][1] * (f_mat[1][0] * f_mat[2][2] - f_mat[1][2] * f_mat[2][0])
        + f_mat[0][2] * (f_mat[1][0] * f_mat[2][1] - f_mat[1][1] * f_mat[2][0])
    )
    meas, lam_v, mu_v = meas_ref[...], lam_ref[...], mu_ref[...]
    alpha = 0.75 * mu_v / lam_v + 1.0
    ic_ver = jnp.maximum(ic + 1.0, 0.0) + 1e-30
    dens = (
        0.5 * mu_v * (ic - 3.0)
        + 0.5 * lam_v * (det - alpha) ** 2
        - 0.5 * mu_v * jnp.log(ic_ver)
    )
    out_ref[0, 0] = jnp.sum(dens * meas)


def _kinetic_body(m_ref, dt_ref, dn_ref, out_ref):
    i = pl.program_id(0)

    @pl.when(i == 0)
    def _():
        out_ref[0, 0] = 0.0

    m = m_ref[...]
    acc = jnp.float32(0.0)
    for k in range(3):
        s = jnp.sum(m * dt_ref[k : k + 1, :], axis=1, keepdims=True)
        acc += jnp.sum(s * dn_ref[:, k : k + 1])
    out_ref[0, 0] += acc


def kernel(position, time_step, state_position, velocity,
           external_acceleration, int_density_matrix, elements, polynomials,
           measure, lam, mu):
    n = position.shape[0]
    e = elements.shape[0]
    eb = e // 128
    f32 = jnp.float32
    dt = jnp.asarray(time_step, f32)
    coeff = 0.5 / (dt * dt)

    table = jnp.pad(position, ((0, 0), (0, _PAD_D - 3)))
    idx_t = elements.T.astype(jnp.int32)
    comp = _sc_gather_components(table, idx_t)

    polyc = jnp.transpose(polynomials, (1, 2, 0)).reshape(16, eb, 128)

    elastic = pl.pallas_call(
        _elastic_body,
        out_shape=jax.ShapeDtypeStruct((1, 1), f32),
        in_specs=[
            pl.BlockSpec((12, eb, 128), lambda: (0, 0, 0)),
            pl.BlockSpec((16, eb, 128), lambda: (0, 0, 0)),
            pl.BlockSpec((eb, 128), lambda: (0, 0)),
            pl.BlockSpec((eb, 128), lambda: (0, 0)),
            pl.BlockSpec((eb, 128), lambda: (0, 0)),
        ],
        out_specs=pl.BlockSpec(memory_space=pltpu.SMEM),
    )(comp, polyc, measure.reshape(eb, 128),
      lam.reshape(eb, 128), mu.reshape(eb, 128))[0, 0]

    delta_t = (position.T - state_position.T - velocity.T * dt
               - external_acceleration.T * (dt * dt)).astype(f32)
    dt_t = jnp.zeros((8, n), f32).at[:3, :].set(delta_t)
    dn = jnp.zeros((n, 8), f32).at[:, :3].set(delta_t.T)

    kin_raw = pl.pallas_call(
        _kinetic_body,
        grid=(n // _ROW_BLOCK,),
        out_shape=jax.ShapeDtypeStruct((1, 1), f32),
        in_specs=[
            pl.BlockSpec((_ROW_BLOCK, n), lambda i: (i, 0)),
            pl.BlockSpec((8, n), lambda i: (0, 0)),
            pl.BlockSpec((_ROW_BLOCK, 8), lambda i: (i, 0)),
        ],
        out_specs=pl.BlockSpec(memory_space=pltpu.SMEM),
    )(int_density_matrix, dt_t, dn)[0, 0]

    return (coeff * kin_raw + elastic).astype(f32)
